# initial kernel scaffold (unmeasured)
import numpy as np
import jax
import jax.numpy as jnp
from jax import lax
from jax.experimental import pallas as pl
from jax.experimental.pallas import tpu as pltpu

N_DEV = 8
SQ = 1024
D = 1024
HQ = 8
DH = 128
SCALE = 0.08838834764831843
BF16 = jnp.bfloat16
F32 = jnp.float32


def _rope_tables():
    inv = 1.0 / (10000.0 ** (np.arange(0, DH, 2) / DH))
    pos = np.arange(SQ)[:, None] * inv[None, :]
    cos = np.repeat(np.cos(pos), 2, axis=-1).astype(np.float32)
    sin = np.repeat(np.sin(pos), 2, axis=-1).astype(np.float32)
    P = np.zeros((DH, DH), np.float32)
    P[np.arange(1, DH, 2), np.arange(0, DH, 2)] = -1.0
    P[np.arange(0, DH, 2), np.arange(1, DH, 2)] = 1.0
    return cos, sin, P.astype(np.float32)


_COS_NP, _SIN_NP, _P_NP = _rope_tables()


def kernel(x, Wq, Wk, Wv, Wo):
    x2 = x[0].astype(BF16)
    wq = Wq.astype(BF16)
    wk = Wk.astype(BF16)
    wv = Wv.astype(BF16)
    wo = Wo.astype(BF16)
    cos = jnp.asarray(_COS_NP)
    sin = jnp.asarray(_SIN_NP)
    rotm = jnp.asarray(_P_NP).astype(BF16)

    def body(x_ref, wq_ref, wk_ref, wv_ref, wo_ref, cos_ref, sin_ref, rot_ref,
             out_ref,
             x_all, pacc, xb_ref, stage, own_ref,
             xsend_sems, xrecv_sems, psend_sems, precv_sems, local_sem):
        me = lax.axis_index("i")

        barrier_sem = pltpu.get_barrier_semaphore()
        for t in range(1, N_DEV):
            peer = (me + t) % N_DEV
            pl.semaphore_signal(barrier_sem, inc=1, device_id=(peer,),
                                device_id_type=pl.DeviceIdType.MESH)
        pl.semaphore_wait(barrier_sem, N_DEV - 1)

        for t in range(1, N_DEV):
            dst = (me + t) % N_DEV
            pltpu.make_async_remote_copy(
                src_ref=x_ref,
                dst_ref=x_all.at[me],
                send_sem=xsend_sems.at[t],
                recv_sem=xrecv_sems.at[me],
                device_id=(dst,),
                device_id_type=pl.DeviceIdType.MESH,
            ).start()

        cos_v = cos_ref[...]
        sin_v = sin_ref[...]
        rot_v = rot_ref[...]

        def attend(xb):
            qf = lax.dot_general(xb, wq_ref[...], (((1,), (0,)), ((), ())),
                                 preferred_element_type=F32)
            kf = lax.dot_general(xb, wk_ref[...], (((1,), (0,)), ((), ())),
                                 preferred_element_type=F32)
            vv = lax.dot_general(xb, wv_ref[...], (((1,), (0,)), ((), ())),
                                 preferred_element_type=F32).astype(BF16)

            def head(h, ctx):
                off = h * DH
                qh = lax.dynamic_slice(qf, (0, off), (SQ, DH))
                kh = lax.dynamic_slice(kf, (0, off), (SQ, DH))
                qr = lax.dot_general(qh.astype(BF16), rot_v,
                                     (((1,), (0,)), ((), ())),
                                     preferred_element_type=F32)
                kr = lax.dot_general(kh.astype(BF16), rot_v,
                                     (((1,), (0,)), ((), ())),
                                     preferred_element_type=F32)
                q = (qh * cos_v + qr * sin_v).astype(BF16)
                k = (kh * cos_v + kr * sin_v).astype(BF16)
                s = lax.dot_general(q, k, (((1,), (1,)), ((), ())),
                                    preferred_element_type=F32) * SCALE
                m = jnp.max(s, axis=1, keepdims=True)
                e = jnp.exp(s - m)
                w = (e / jnp.sum(e, axis=1, keepdims=True)).astype(BF16)
                vh = lax.dynamic_slice(vv, (0, off), (SQ, DH))
                c = lax.dot_general(w, vh, (((1,), (0,)), ((), ())),
                                    preferred_element_type=F32).astype(BF16)
                return lax.dynamic_update_slice(ctx, c, (0, off))

            ctx = lax.fori_loop(0, HQ, head, jnp.zeros((SQ, D), BF16))
            return lax.dot_general(ctx, wo_ref[...], (((1,), (0,)), ((), ())),
                                   preferred_element_type=F32)

        for t in range(N_DEV):
            b = (me + t) % N_DEV
            if t == 0:
                xb = x_ref[...]
            else:
                pltpu.make_async_remote_copy(
                    src_ref=x_ref, dst_ref=x_all.at[b],
                    send_sem=xsend_sems.at[t], recv_sem=xrecv_sems.at[b],
                    device_id=(b,), device_id_type=pl.DeviceIdType.MESH,
                ).wait_recv()
                cp = pltpu.make_async_copy(x_all.at[b], xb_ref, local_sem)
                cp.start()
                cp.wait()
                xb = xb_ref[...]

            partial = attend(xb)

            if t == 0:
                own_ref[...] = partial.astype(BF16)
            else:
                slot = t % 2
                if t >= 3:
                    pltpu.make_async_remote_copy(
                        src_ref=stage.at[slot], dst_ref=pacc.at[me],
                        send_sem=psend_sems.at[slot], recv_sem=precv_sems.at[me],
                        device_id=(b,), device_id_type=pl.DeviceIdType.MESH,
                    ).wait_send()
                stage[slot] = partial.astype(BF16)
                pltpu.make_async_remote_copy(
                    src_ref=stage.at[slot], dst_ref=pacc.at[me],
                    send_sem=psend_sems.at[slot], recv_sem=precv_sems.at[me],
                    device_id=(b,), device_id_type=pl.DeviceIdType.MESH,
                ).start()

        for t in (N_DEV - 2, N_DEV - 1):
            pltpu.make_async_remote_copy(
                src_ref=stage.at[t % 2], dst_ref=pacc.at[me],
                send_sem=psend_sems.at[t % 2], recv_sem=precv_sems.at[me],
                device_id=((me + t) % N_DEV,),
                device_id_type=pl.DeviceIdType.MESH,
            ).wait_send()
        for t in range(1, N_DEV):
            pltpu.make_async_remote_copy(
                src_ref=x_ref, dst_ref=x_all.at[me],
                send_sem=xsend_sems.at[t], recv_sem=xrecv_sems.at[me],
                device_id=((me + t) % N_DEV,),
                device_id_type=pl.DeviceIdType.MESH,
            ).wait_send()

        acc = own_ref[...].astype(F32)
        for t in range(1, N_DEV):
            src = (me + t) % N_DEV
            pltpu.make_async_remote_copy(
                src_ref=x_ref, dst_ref=pacc.at[src],
                send_sem=xsend_sems.at[t], recv_sem=precv_sems.at[src],
                device_id=(src,), device_id_type=pl.DeviceIdType.MESH,
            ).wait_recv()
            cp = pltpu.make_async_copy(pacc.at[src], xb_ref, local_sem)
            cp.start()
            cp.wait()
            acc = acc + xb_ref[...].astype(F32)
        out_ref[0] = acc.astype(BF16)

    out = pl.pallas_call(
        body,
        out_shape=jax.ShapeDtypeStruct((1, SQ, D), BF16),
        in_specs=[pl.BlockSpec(memory_space=pltpu.VMEM)] * 8,
        out_specs=pl.BlockSpec(memory_space=pltpu.VMEM),
        scratch_shapes=[
            pl.ANY((N_DEV, SQ, D), BF16),
            pl.ANY((N_DEV, SQ, D), BF16),
            pltpu.VMEM((SQ, D), BF16),
            pltpu.VMEM((2, SQ, D), BF16),
            pltpu.VMEM((SQ, D), BF16),
            pltpu.SemaphoreType.DMA((N_DEV,)),
            pltpu.SemaphoreType.DMA((N_DEV,)),
            pltpu.SemaphoreType.DMA((2,)),
            pltpu.SemaphoreType.DMA((N_DEV,)),
            pltpu.SemaphoreType.DMA,
        ],
        compiler_params=pltpu.CompilerParams(collective_id=0),
    )(x2, wq, wk, wv, wo, cos, sin, rotm)
    return out


# baseline (device time: 446617 ns/iter reference)
import numpy as np
import jax
import jax.numpy as jnp
from jax import lax
from jax.experimental import pallas as pl
from jax.experimental.pallas import tpu as pltpu

N_DEV = 8
SQ = 1024
D = 1024
HQ = 8
DH = 128
SCALE = 0.08838834764831843
BF16 = jnp.bfloat16
F32 = jnp.float32


def _rope_tables():
    inv = 1.0 / (10000.0 ** (np.arange(0, DH, 2) / DH))
    pos = np.arange(SQ)[:, None] * inv[None, :]
    cos = np.repeat(np.cos(pos), 2, axis=-1).astype(np.float32)
    sin = np.repeat(np.sin(pos), 2, axis=-1).astype(np.float32)
    P = np.zeros((DH, DH), np.float32)
    P[np.arange(1, DH, 2), np.arange(0, DH, 2)] = -1.0
    P[np.arange(0, DH, 2), np.arange(1, DH, 2)] = 1.0
    return cos, sin, P.astype(np.float32)


_COS_NP, _SIN_NP, _P_NP = _rope_tables()


def kernel(x, Wq, Wk, Wv, Wo):
    x2 = x[0].astype(BF16)
    wq = Wq.astype(BF16)
    wk = Wk.astype(BF16)
    wv = Wv.astype(BF16)
    wo = Wo.astype(BF16)
    cos = jnp.asarray(_COS_NP)
    sin = jnp.asarray(_SIN_NP)
    rotm = jnp.asarray(_P_NP).astype(BF16)

    def body(x_ref, wq_ref, wk_ref, wv_ref, wo_ref, cos_ref, sin_ref, rot_ref,
             out_ref, x_all, pacc,
             xb_ref, stage, own_ref, ctx_ref, qf_ref, kf_ref, v_ref,
             xsend_sems, xrecv_sems, psend_sems, precv_sems, local_sem):
        me = lax.axis_index("i")

        barrier_sem = pltpu.get_barrier_semaphore()
        for t in range(1, N_DEV):
            peer = (me + t) % N_DEV
            pl.semaphore_signal(barrier_sem, inc=1, device_id=(peer,),
                                device_id_type=pl.DeviceIdType.MESH)
        pl.semaphore_wait(barrier_sem, N_DEV - 1)

        for t in range(1, N_DEV):
            dst = (me + t) % N_DEV
            pltpu.make_async_remote_copy(
                src_ref=x_ref,
                dst_ref=x_all.at[me],
                send_sem=xsend_sems.at[t],
                recv_sem=xrecv_sems.at[me],
                device_id=(dst,),
                device_id_type=pl.DeviceIdType.MESH,
            ).start()

        cos_v = cos_ref[...]
        sin_v = sin_ref[...]
        rot_v = rot_ref[...]

        def head(h, carry):
            off = pl.multiple_of(h * DH, DH)
            qh = qf_ref[:, pl.ds(off, DH)]
            kh = kf_ref[:, pl.ds(off, DH)]
            qr = lax.dot_general(qh.astype(BF16), rot_v,
                                 (((1,), (0,)), ((), ())),
                                 preferred_element_type=F32)
            kr = lax.dot_general(kh.astype(BF16), rot_v,
                                 (((1,), (0,)), ((), ())),
                                 preferred_element_type=F32)
            q = (qh * cos_v + qr * sin_v).astype(BF16)
            k = (kh * cos_v + kr * sin_v).astype(BF16)
            s = lax.dot_general(q, k, (((1,), (1,)), ((), ())),
                                preferred_element_type=F32) * SCALE
            m = jnp.max(s, axis=1, keepdims=True)
            e = jnp.exp(s - m)
            w = (e / jnp.sum(e, axis=1, keepdims=True)).astype(BF16)
            vh = v_ref[:, pl.ds(off, DH)]
            c = lax.dot_general(w, vh, (((1,), (0,)), ((), ())),
                                preferred_element_type=F32).astype(BF16)
            ctx_ref[:, pl.ds(off, DH)] = c
            return carry

        def attend(xb):
            qf_ref[...] = lax.dot_general(xb, wq_ref[...], (((1,), (0,)), ((), ())),
                                          preferred_element_type=F32)
            kf_ref[...] = lax.dot_general(xb, wk_ref[...], (((1,), (0,)), ((), ())),
                                          preferred_element_type=F32)
            v_ref[...] = lax.dot_general(xb, wv_ref[...], (((1,), (0,)), ((), ())),
                                         preferred_element_type=F32).astype(BF16)
            lax.fori_loop(0, HQ, head, 0)
            return lax.dot_general(ctx_ref[...], wo_ref[...],
                                   (((1,), (0,)), ((), ())),
                                   preferred_element_type=F32)

        for t in range(N_DEV):
            b = (me + t) % N_DEV
            if t == 0:
                xb = x_ref[...]
            else:
                pltpu.make_async_remote_copy(
                    src_ref=x_ref, dst_ref=x_all.at[b],
                    send_sem=xsend_sems.at[t], recv_sem=xrecv_sems.at[b],
                    device_id=(b,), device_id_type=pl.DeviceIdType.MESH,
                ).wait_recv()
                cp = pltpu.make_async_copy(x_all.at[b], xb_ref, local_sem)
                cp.start()
                cp.wait()
                xb = xb_ref[...]

            partial = attend(xb)

            if t == 0:
                own_ref[...] = partial.astype(BF16)
            else:
                slot = t % 2
                if t >= 3:
                    pltpu.make_async_remote_copy(
                        src_ref=stage.at[slot], dst_ref=pacc.at[me],
                        send_sem=psend_sems.at[slot], recv_sem=precv_sems.at[me],
                        device_id=(b,), device_id_type=pl.DeviceIdType.MESH,
                    ).wait_send()
                stage[slot] = partial.astype(BF16)
                pltpu.make_async_remote_copy(
                    src_ref=stage.at[slot], dst_ref=pacc.at[me],
                    send_sem=psend_sems.at[slot], recv_sem=precv_sems.at[me],
                    device_id=(b,), device_id_type=pl.DeviceIdType.MESH,
                ).start()

        for t in (N_DEV - 2, N_DEV - 1):
            pltpu.make_async_remote_copy(
                src_ref=stage.at[t % 2], dst_ref=pacc.at[me],
                send_sem=psend_sems.at[t % 2], recv_sem=precv_sems.at[me],
                device_id=((me + t) % N_DEV,),
                device_id_type=pl.DeviceIdType.MESH,
            ).wait_send()
        for t in range(1, N_DEV):
            pltpu.make_async_remote_copy(
                src_ref=x_ref, dst_ref=x_all.at[me],
                send_sem=xsend_sems.at[t], recv_sem=xrecv_sems.at[me],
                device_id=((me + t) % N_DEV,),
                device_id_type=pl.DeviceIdType.MESH,
            ).wait_send()

        acc = own_ref[...].astype(F32)
        for t in range(1, N_DEV):
            src = (me + t) % N_DEV
            pltpu.make_async_remote_copy(
                src_ref=x_ref, dst_ref=pacc.at[src],
                send_sem=xsend_sems.at[t], recv_sem=precv_sems.at[src],
                device_id=(src,), device_id_type=pl.DeviceIdType.MESH,
            ).wait_recv()
            cp = pltpu.make_async_copy(pacc.at[src], xb_ref, local_sem)
            cp.start()
            cp.wait()
            acc = acc + xb_ref[...].astype(F32)
        out_ref[0] = acc.astype(BF16)

    out, _, _ = pl.pallas_call(
        body,
        out_shape=[
            jax.ShapeDtypeStruct((1, SQ, D), BF16),
            jax.ShapeDtypeStruct((N_DEV, SQ, D), BF16),
            jax.ShapeDtypeStruct((N_DEV, SQ, D), BF16),
        ],
        in_specs=[pl.BlockSpec(memory_space=pltpu.VMEM)] * 8,
        out_specs=[
            pl.BlockSpec(memory_space=pltpu.VMEM),
            pl.BlockSpec(memory_space=pl.ANY),
            pl.BlockSpec(memory_space=pl.ANY),
        ],
        scratch_shapes=[
            pltpu.VMEM((SQ, D), BF16),
            pltpu.VMEM((2, SQ, D), BF16),
            pltpu.VMEM((SQ, D), BF16),
            pltpu.VMEM((SQ, D), BF16),
            pltpu.VMEM((SQ, D), F32),
            pltpu.VMEM((SQ, D), F32),
            pltpu.VMEM((SQ, D), BF16),
            pltpu.SemaphoreType.DMA((N_DEV,)),
            pltpu.SemaphoreType.DMA((N_DEV,)),
            pltpu.SemaphoreType.DMA((2,)),
            pltpu.SemaphoreType.DMA((N_DEV,)),
            pltpu.SemaphoreType.DMA,
        ],
        compiler_params=pltpu.CompilerParams(collective_id=0),
    )(x2, wq, wk, wv, wo, cos, sin, rotm)
    return out


# device time: 395211 ns/iter; 1.1301x vs baseline; 1.1301x over previous
import numpy as np
import jax
import jax.numpy as jnp
from jax import lax
from jax.experimental import pallas as pl
from jax.experimental.pallas import tpu as pltpu

N_DEV = 8
SQ = 1024
D = 1024
HQ = 8
DH = 128
SCALE = 0.08838834764831843
BF16 = jnp.bfloat16
F32 = jnp.float32


def _rope_tables():
    inv = 1.0 / (10000.0 ** (np.arange(0, DH, 2) / DH))
    pos = np.arange(SQ)[:, None] * inv[None, :]
    cos = np.repeat(np.cos(pos), 2, axis=-1).astype(np.float32)
    sin = np.repeat(np.sin(pos), 2, axis=-1).astype(np.float32)
    P = np.zeros((DH, DH), np.float32)
    P[np.arange(1, DH, 2), np.arange(0, DH, 2)] = -1.0
    P[np.arange(0, DH, 2), np.arange(1, DH, 2)] = 1.0
    return cos, sin, P.astype(np.float32)


_COS_NP, _SIN_NP, _P_NP = _rope_tables()


def kernel(x, Wq, Wk, Wv, Wo):
    x2 = x[0].astype(BF16)
    wq = Wq.astype(BF16)
    wk = Wk.astype(BF16)
    wv = Wv.astype(BF16)
    wo = Wo.astype(BF16)
    cos = jnp.asarray(_COS_NP)
    sin = jnp.asarray(_SIN_NP)
    rotm = jnp.asarray(_P_NP).astype(BF16)

    def body(x_ref, wq_ref, wk_ref, wv_ref, wo_ref, cos_ref, sin_ref, rot_ref,
             out_ref, x_all, pacc,
             xb_ref, stage, pb_ref, ctx_ref, qf_ref, kf_ref, v_ref, acc_ref,
             xsend_sems, xrecv_sems, psend_sems, precv_sems, local_sem):
        me = lax.axis_index("i")

        barrier_sem = pltpu.get_barrier_semaphore()
        for t in range(1, N_DEV):
            peer = (me + t) % N_DEV
            pl.semaphore_signal(barrier_sem, inc=1, device_id=(peer,),
                                device_id_type=pl.DeviceIdType.MESH)
        pl.semaphore_wait(barrier_sem, N_DEV - 1)

        for t in range(1, N_DEV):
            dst = (me + t) % N_DEV
            pltpu.make_async_remote_copy(
                src_ref=x_ref,
                dst_ref=x_all.at[me],
                send_sem=xsend_sems.at[t],
                recv_sem=xrecv_sems.at[me],
                device_id=(dst,),
                device_id_type=pl.DeviceIdType.MESH,
            ).start()

        cos_v = cos_ref[...]
        sin_v = sin_ref[...]
        rot_v = rot_ref[...]

        def head(h, carry):
            off = pl.multiple_of(h * DH, DH)
            qh = qf_ref[:, pl.ds(off, DH)]
            kh = kf_ref[:, pl.ds(off, DH)]
            qr = lax.dot_general(qh.astype(BF16), rot_v,
                                 (((1,), (0,)), ((), ())),
                                 preferred_element_type=F32)
            kr = lax.dot_general(kh.astype(BF16), rot_v,
                                 (((1,), (0,)), ((), ())),
                                 preferred_element_type=F32)
            q = (qh * cos_v + qr * sin_v).astype(BF16)
            k = (kh * cos_v + kr * sin_v).astype(BF16)
            s = lax.dot_general(q, k, (((1,), (1,)), ((), ())),
                                preferred_element_type=F32) * SCALE
            m = jnp.max(s, axis=1, keepdims=True)
            e = jnp.exp(s - m)
            w = (e / jnp.sum(e, axis=1, keepdims=True)).astype(BF16)
            vh = v_ref[:, pl.ds(off, DH)]
            c = lax.dot_general(w, vh, (((1,), (0,)), ((), ())),
                                preferred_element_type=F32).astype(BF16)
            ctx_ref[:, pl.ds(off, DH)] = c
            return carry

        def attend(xb):
            qf_ref[...] = lax.dot_general(xb, wq_ref[...], (((1,), (0,)), ((), ())),
                                          preferred_element_type=F32)
            kf_ref[...] = lax.dot_general(xb, wk_ref[...], (((1,), (0,)), ((), ())),
                                          preferred_element_type=F32)
            v_ref[...] = lax.dot_general(xb, wv_ref[...], (((1,), (0,)), ((), ())),
                                         preferred_element_type=F32).astype(BF16)
            lax.fori_loop(0, HQ, head, 0)
            return lax.dot_general(ctx_ref[...], wo_ref[...],
                                   (((1,), (0,)), ((), ())),
                                   preferred_element_type=F32)

        def accumulate(src):
            pltpu.make_async_remote_copy(
                src_ref=x_ref, dst_ref=pacc.at[src],
                send_sem=psend_sems.at[0], recv_sem=precv_sems.at[src],
                device_id=(src,), device_id_type=pl.DeviceIdType.MESH,
            ).wait_recv()
            cp = pltpu.make_async_copy(pacc.at[src], pb_ref, local_sem)
            cp.start()
            cp.wait()
            acc_ref[...] = acc_ref[...] + pb_ref[...].astype(F32)

        for t in range(N_DEV):
            b = (me - t) % N_DEV
            if t == 0:
                xb = x_ref[...]
            else:
                pltpu.make_async_remote_copy(
                    src_ref=x_ref, dst_ref=x_all.at[b],
                    send_sem=xsend_sems.at[t], recv_sem=xrecv_sems.at[b],
                    device_id=(b,), device_id_type=pl.DeviceIdType.MESH,
                ).wait_recv()
                cp = pltpu.make_async_copy(x_all.at[b], xb_ref, local_sem)
                cp.start()
                cp.wait()
                xb = xb_ref[...]

            partial = attend(xb)

            if t == 0:
                acc_ref[...] = partial
            else:
                slot = t % 2
                if t >= 3:
                    pltpu.make_async_remote_copy(
                        src_ref=stage.at[slot], dst_ref=pacc.at[me],
                        send_sem=psend_sems.at[slot], recv_sem=precv_sems.at[me],
                        device_id=(b,), device_id_type=pl.DeviceIdType.MESH,
                    ).wait_send()
                stage[slot] = partial.astype(BF16)
                pltpu.make_async_remote_copy(
                    src_ref=stage.at[slot], dst_ref=pacc.at[me],
                    send_sem=psend_sems.at[slot], recv_sem=precv_sems.at[me],
                    device_id=(b,), device_id_type=pl.DeviceIdType.MESH,
                ).start()

            if t >= 2:
                accumulate((me + t - 1) % N_DEV)

        accumulate((me + N_DEV - 1) % N_DEV)
        out_ref[0] = acc_ref[...].astype(BF16)

        for t in (N_DEV - 2, N_DEV - 1):
            pltpu.make_async_remote_copy(
                src_ref=stage.at[t % 2], dst_ref=pacc.at[me],
                send_sem=psend_sems.at[t % 2], recv_sem=precv_sems.at[me],
                device_id=((me - t) % N_DEV,),
                device_id_type=pl.DeviceIdType.MESH,
            ).wait_send()
        for t in range(1, N_DEV):
            pltpu.make_async_remote_copy(
                src_ref=x_ref, dst_ref=x_all.at[me],
                send_sem=xsend_sems.at[t], recv_sem=xrecv_sems.at[me],
                device_id=((me + t) % N_DEV,),
                device_id_type=pl.DeviceIdType.MESH,
            ).wait_send()

    out, _, _ = pl.pallas_call(
        body,
        out_shape=[
            jax.ShapeDtypeStruct((1, SQ, D), BF16),
            jax.ShapeDtypeStruct((N_DEV, SQ, D), BF16),
            jax.ShapeDtypeStruct((N_DEV, SQ, D), BF16),
        ],
        in_specs=[pl.BlockSpec(memory_space=pltpu.VMEM)] * 8,
        out_specs=[
            pl.BlockSpec(memory_space=pltpu.VMEM),
            pl.BlockSpec(memory_space=pl.ANY),
            pl.BlockSpec(memory_space=pl.ANY),
        ],
        scratch_shapes=[
            pltpu.VMEM((SQ, D), BF16),
            pltpu.VMEM((2, SQ, D), BF16),
            pltpu.VMEM((SQ, D), BF16),
            pltpu.VMEM((SQ, D), BF16),
            pltpu.VMEM((SQ, D), F32),
            pltpu.VMEM((SQ, D), F32),
            pltpu.VMEM((SQ, D), BF16),
            pltpu.VMEM((SQ, D), F32),
            pltpu.SemaphoreType.DMA((N_DEV,)),
            pltpu.SemaphoreType.DMA((N_DEV,)),
            pltpu.SemaphoreType.DMA((2,)),
            pltpu.SemaphoreType.DMA((N_DEV,)),
            pltpu.SemaphoreType.DMA,
        ],
        compiler_params=pltpu.CompilerParams(
            collective_id=0, vmem_limit_bytes=40 * 1024 * 1024),
    )(x2, wq, wk, wv, wo, cos, sin, rotm)
    return out


# device time: 363126 ns/iter; 1.2299x vs baseline; 1.0884x over previous
import numpy as np
import jax
import jax.numpy as jnp
from jax import lax
from jax.experimental import pallas as pl
from jax.experimental.pallas import tpu as pltpu

N_DEV = 8
SQ = 1024
D = 1024
HQ = 8
DH = 128
SCALE = 0.08838834764831843
BF16 = jnp.bfloat16
F32 = jnp.float32


def _rope_tables():
    inv = 1.0 / (10000.0 ** (np.arange(0, DH, 2) / DH))
    pos = np.arange(SQ)[:, None] * inv[None, :]
    cos = np.repeat(np.cos(pos), 2, axis=-1).astype(np.float32)
    sin = np.repeat(np.sin(pos), 2, axis=-1).astype(np.float32)
    P = np.zeros((DH, DH), np.float32)
    P[np.arange(1, DH, 2), np.arange(0, DH, 2)] = -1.0
    P[np.arange(0, DH, 2), np.arange(1, DH, 2)] = 1.0
    return cos, sin, P.astype(np.float32)


_COS_NP, _SIN_NP, _P_NP = _rope_tables()


def kernel(x, Wq, Wk, Wv, Wo):
    x2 = x[0].astype(BF16)
    wq = Wq.astype(BF16)
    wk = Wk.astype(BF16)
    wv = Wv.astype(BF16)
    wo = Wo.astype(BF16)
    cos = jnp.asarray(_COS_NP)
    sin = jnp.asarray(_SIN_NP)
    rotm = jnp.asarray(_P_NP).astype(BF16)

    def body(x_ref, wq_ref, wk_ref, wv_ref, wo_ref, cos_ref, sin_ref, rot_ref,
             out_ref, x_all, pacc,
             xb_ref, stage, pb_ref, ctx_ref, qf_ref, kf_ref, v_ref, acc_ref,
             xsend_sems, xrecv_sems, psend_sems, precv_sems, local_sem):
        me = lax.axis_index("i")

        barrier_sem = pltpu.get_barrier_semaphore()
        for t in range(1, N_DEV):
            peer = (me + t) % N_DEV
            pl.semaphore_signal(barrier_sem, inc=1, device_id=(peer,),
                                device_id_type=pl.DeviceIdType.MESH)
        pl.semaphore_wait(barrier_sem, N_DEV - 1)

        for t in range(1, N_DEV):
            dst = (me + t) % N_DEV
            pltpu.make_async_remote_copy(
                src_ref=x_ref,
                dst_ref=x_all.at[me],
                send_sem=xsend_sems.at[t],
                recv_sem=xrecv_sems.at[me],
                device_id=(dst,),
                device_id_type=pl.DeviceIdType.MESH,
            ).start()

        cos_v = cos_ref[...]
        sin_v = sin_ref[...]
        rot_v = rot_ref[...]

        even = (lax.broadcasted_iota(jnp.int32, (SQ, DH), 1) % 2) == 0
        ones_col = jnp.full((SQ, 1), 1.0, BF16)

        def rot(t):
            return jnp.where(even, -pltpu.roll(t, DH - 1, 1), pltpu.roll(t, 1, 1))

        def head(h, carry):
            off = pl.multiple_of(h * DH, DH)
            qh = qf_ref[:, pl.ds(off, DH)]
            kh = kf_ref[:, pl.ds(off, DH)]
            q = ((qh * cos_v + rot(qh) * sin_v) * SCALE).astype(BF16)
            k = (kh * cos_v + rot(kh) * sin_v).astype(BF16)
            s = lax.dot_general(q, k, (((1,), (1,)), ((), ())),
                                preferred_element_type=F32)
            e16 = jnp.exp(s.astype(BF16))
            l = lax.dot_general(e16, ones_col, (((1,), (0,)), ((), ())),
                                preferred_element_type=F32)
            vh = v_ref[:, pl.ds(off, DH)]
            cu = lax.dot_general(e16, vh, (((1,), (0,)), ((), ())),
                                 preferred_element_type=F32)
            c = cu * (1.0 / l)
            ctx_ref[:, pl.ds(off, DH)] = c.astype(BF16)
            return carry

        def attend(xb):
            qf_ref[...] = lax.dot_general(xb, wq_ref[...], (((1,), (0,)), ((), ())),
                                          preferred_element_type=F32)
            kf_ref[...] = lax.dot_general(xb, wk_ref[...], (((1,), (0,)), ((), ())),
                                          preferred_element_type=F32)
            v_ref[...] = lax.dot_general(xb, wv_ref[...], (((1,), (0,)), ((), ())),
                                         preferred_element_type=F32).astype(BF16)
            lax.fori_loop(0, HQ, head, 0)
            return lax.dot_general(ctx_ref[...], wo_ref[...],
                                   (((1,), (0,)), ((), ())),
                                   preferred_element_type=F32)

        def accumulate(src):
            pltpu.make_async_remote_copy(
                src_ref=x_ref, dst_ref=pacc.at[src],
                send_sem=psend_sems.at[0], recv_sem=precv_sems.at[src],
                device_id=(src,), device_id_type=pl.DeviceIdType.MESH,
            ).wait_recv()
            cp = pltpu.make_async_copy(pacc.at[src], pb_ref, local_sem)
            cp.start()
            cp.wait()
            acc_ref[...] = acc_ref[...] + pb_ref[...].astype(F32)

        for t in range(N_DEV):
            b = (me - t) % N_DEV
            if t == 0:
                xb = x_ref[...]
            else:
                pltpu.make_async_remote_copy(
                    src_ref=x_ref, dst_ref=x_all.at[b],
                    send_sem=xsend_sems.at[t], recv_sem=xrecv_sems.at[b],
                    device_id=(b,), device_id_type=pl.DeviceIdType.MESH,
                ).wait_recv()
                cp = pltpu.make_async_copy(x_all.at[b], xb_ref, local_sem)
                cp.start()
                cp.wait()
                xb = xb_ref[...]

            partial = attend(xb)

            if t == 0:
                acc_ref[...] = partial
            else:
                slot = t % 2
                if t >= 3:
                    pltpu.make_async_remote_copy(
                        src_ref=stage.at[slot], dst_ref=pacc.at[me],
                        send_sem=psend_sems.at[slot], recv_sem=precv_sems.at[me],
                        device_id=(b,), device_id_type=pl.DeviceIdType.MESH,
                    ).wait_send()
                stage[slot] = partial.astype(BF16)
                pltpu.make_async_remote_copy(
                    src_ref=stage.at[slot], dst_ref=pacc.at[me],
                    send_sem=psend_sems.at[slot], recv_sem=precv_sems.at[me],
                    device_id=(b,), device_id_type=pl.DeviceIdType.MESH,
                ).start()

            if t >= 2:
                accumulate((me + t - 1) % N_DEV)

        accumulate((me + N_DEV - 1) % N_DEV)
        out_ref[0] = acc_ref[...].astype(BF16)

        for t in (N_DEV - 2, N_DEV - 1):
            pltpu.make_async_remote_copy(
                src_ref=stage.at[t % 2], dst_ref=pacc.at[me],
                send_sem=psend_sems.at[t % 2], recv_sem=precv_sems.at[me],
                device_id=((me - t) % N_DEV,),
                device_id_type=pl.DeviceIdType.MESH,
            ).wait_send()
        for t in range(1, N_DEV):
            pltpu.make_async_remote_copy(
                src_ref=x_ref, dst_ref=x_all.at[me],
                send_sem=xsend_sems.at[t], recv_sem=xrecv_sems.at[me],
                device_id=((me + t) % N_DEV,),
                device_id_type=pl.DeviceIdType.MESH,
            ).wait_send()

    out, _, _ = pl.pallas_call(
        body,
        out_shape=[
            jax.ShapeDtypeStruct((1, SQ, D), BF16),
            jax.ShapeDtypeStruct((N_DEV, SQ, D), BF16),
            jax.ShapeDtypeStruct((N_DEV, SQ, D), BF16),
        ],
        in_specs=[pl.BlockSpec(memory_space=pltpu.VMEM)] * 8,
        out_specs=[
            pl.BlockSpec(memory_space=pltpu.VMEM),
            pl.BlockSpec(memory_space=pl.ANY),
            pl.BlockSpec(memory_space=pl.ANY),
        ],
        scratch_shapes=[
            pltpu.VMEM((SQ, D), BF16),
            pltpu.VMEM((2, SQ, D), BF16),
            pltpu.VMEM((SQ, D), BF16),
            pltpu.VMEM((SQ, D), BF16),
            pltpu.VMEM((SQ, D), F32),
            pltpu.VMEM((SQ, D), F32),
            pltpu.VMEM((SQ, D), BF16),
            pltpu.VMEM((SQ, D), F32),
            pltpu.SemaphoreType.DMA((N_DEV,)),
            pltpu.SemaphoreType.DMA((N_DEV,)),
            pltpu.SemaphoreType.DMA((2,)),
            pltpu.SemaphoreType.DMA((N_DEV,)),
            pltpu.SemaphoreType.DMA,
        ],
        compiler_params=pltpu.CompilerParams(
            collective_id=0, vmem_limit_bytes=40 * 1024 * 1024),
    )(x2, wq, wk, wv, wo, cos, sin, rotm)
    return out


# device time: 360568 ns/iter; 1.2386x vs baseline; 1.0071x over previous
import numpy as np
import jax
import jax.numpy as jnp
from jax import lax
from jax.experimental import pallas as pl
from jax.experimental.pallas import tpu as pltpu

N_DEV = 8
SQ = 1024
D = 1024
HQ = 8
DH = 128
SCALE = 0.08838834764831843
BF16 = jnp.bfloat16
F32 = jnp.float32


def _rope_tables():
    inv = 1.0 / (10000.0 ** (np.arange(0, DH, 2) / DH))
    pos = np.arange(SQ)[:, None] * inv[None, :]
    cos = np.repeat(np.cos(pos), 2, axis=-1).astype(np.float32)
    sin = np.repeat(np.sin(pos), 2, axis=-1).astype(np.float32)
    P = np.zeros((DH, DH), np.float32)
    P[np.arange(1, DH, 2), np.arange(0, DH, 2)] = -1.0
    P[np.arange(0, DH, 2), np.arange(1, DH, 2)] = 1.0
    return cos, sin, P.astype(np.float32)


_COS_NP, _SIN_NP, _P_NP = _rope_tables()


def kernel(x, Wq, Wk, Wv, Wo):
    x2 = x[0].astype(BF16)
    wq = Wq.astype(BF16)
    wk = Wk.astype(BF16)
    wv = Wv.astype(BF16)
    wo = Wo.astype(BF16)
    cos = jnp.asarray(_COS_NP)
    sin = jnp.asarray(_SIN_NP)

    def body(x_ref, wq_ref, wk_ref, wv_ref, wo_ref, cos_ref, sin_ref,
             out_ref, x_all, pacc,
             xb_ref, stage, pb_ref, ctx_ref, qf_ref, kf_ref, v_ref, acc_ref,
             xsend_sems, xrecv_sems, psend_sems, precv_sems, local_sem):
        me = lax.axis_index("i")

        barrier_sem = pltpu.get_barrier_semaphore()
        for t in range(1, N_DEV):
            peer = (me + t) % N_DEV
            pl.semaphore_signal(barrier_sem, inc=1, device_id=(peer,),
                                device_id_type=pl.DeviceIdType.MESH)
        pl.semaphore_wait(barrier_sem, N_DEV - 1)

        for t in range(1, N_DEV):
            dst = (me + t) % N_DEV
            pltpu.make_async_remote_copy(
                src_ref=x_ref,
                dst_ref=x_all.at[me],
                send_sem=xsend_sems.at[t],
                recv_sem=xrecv_sems.at[me],
                device_id=(dst,),
                device_id_type=pl.DeviceIdType.MESH,
            ).start()

        cos_v = cos_ref[...]
        sin_v = sin_ref[...]

        even = (lax.broadcasted_iota(jnp.int32, (SQ, DH), 1) % 2) == 0
        ones_col = jnp.full((SQ, 1), 1.0, BF16)

        def rot(t):
            return jnp.where(even, -pltpu.roll(t, DH - 1, 1), pltpu.roll(t, 1, 1))

        def head(h, carry):
            off = pl.multiple_of(h * DH, DH)
            qh = qf_ref[:, pl.ds(off, DH)]
            kh = kf_ref[:, pl.ds(off, DH)]
            q = ((qh * cos_v + rot(qh) * sin_v) * SCALE).astype(BF16)
            k = (kh * cos_v + rot(kh) * sin_v).astype(BF16)
            s = lax.dot_general(q, k, (((1,), (1,)), ((), ())),
                                preferred_element_type=F32)
            e16 = jnp.exp(s.astype(BF16))
            l = lax.dot_general(e16, ones_col, (((1,), (0,)), ((), ())),
                                preferred_element_type=F32)
            vh = v_ref[:, pl.ds(off, DH)]
            cu = lax.dot_general(e16, vh, (((1,), (0,)), ((), ())),
                                 preferred_element_type=F32)
            c = cu * (1.0 / l)
            ctx_ref[:, pl.ds(off, DH)] = c.astype(BF16)
            return carry

        def attend(xb):
            qf_ref[...] = lax.dot_general(xb, wq_ref[...], (((1,), (0,)), ((), ())),
                                          preferred_element_type=F32)
            kf_ref[...] = lax.dot_general(xb, wk_ref[...], (((1,), (0,)), ((), ())),
                                          preferred_element_type=F32)
            v_ref[...] = lax.dot_general(xb, wv_ref[...], (((1,), (0,)), ((), ())),
                                         preferred_element_type=F32).astype(BF16)
            lax.fori_loop(0, HQ, head, 0)
            return lax.dot_general(ctx_ref[...], wo_ref[...],
                                   (((1,), (0,)), ((), ())),
                                   preferred_element_type=F32)

        def accumulate(src):
            pltpu.make_async_remote_copy(
                src_ref=x_ref, dst_ref=pacc.at[src],
                send_sem=psend_sems.at[0], recv_sem=precv_sems.at[src],
                device_id=(src,), device_id_type=pl.DeviceIdType.MESH,
            ).wait_recv()
            cp = pltpu.make_async_copy(pacc.at[src], pb_ref, local_sem)
            cp.start()
            cp.wait()
            acc_ref[...] = acc_ref[...] + pb_ref[...].astype(F32)

        for t in range(N_DEV):
            b = (me - t) % N_DEV
            if t == 0:
                xb = x_ref[...]
            else:
                pltpu.make_async_remote_copy(
                    src_ref=x_ref, dst_ref=x_all.at[b],
                    send_sem=xsend_sems.at[t], recv_sem=xrecv_sems.at[b],
                    device_id=(b,), device_id_type=pl.DeviceIdType.MESH,
                ).wait_recv()
                cp = pltpu.make_async_copy(x_all.at[b], xb_ref, local_sem)
                cp.start()
                cp.wait()
                xb = xb_ref[...]

            partial = attend(xb)

            if t == 0:
                acc_ref[...] = partial
            else:
                slot = t % 2
                if t >= 3:
                    pltpu.make_async_remote_copy(
                        src_ref=stage.at[slot], dst_ref=pacc.at[me],
                        send_sem=psend_sems.at[slot], recv_sem=precv_sems.at[me],
                        device_id=(b,), device_id_type=pl.DeviceIdType.MESH,
                    ).wait_send()
                stage[slot] = partial.astype(BF16)
                pltpu.make_async_remote_copy(
                    src_ref=stage.at[slot], dst_ref=pacc.at[me],
                    send_sem=psend_sems.at[slot], recv_sem=precv_sems.at[me],
                    device_id=(b,), device_id_type=pl.DeviceIdType.MESH,
                ).start()

            if t >= 2:
                accumulate((me + t - 1) % N_DEV)

        accumulate((me + N_DEV - 1) % N_DEV)
        out_ref[0] = acc_ref[...].astype(BF16)

        for t in (N_DEV - 2, N_DEV - 1):
            pltpu.make_async_remote_copy(
                src_ref=stage.at[t % 2], dst_ref=pacc.at[me],
                send_sem=psend_sems.at[t % 2], recv_sem=precv_sems.at[me],
                device_id=((me - t) % N_DEV,),
                device_id_type=pl.DeviceIdType.MESH,
            ).wait_send()
        for t in range(1, N_DEV):
            pltpu.make_async_remote_copy(
                src_ref=x_ref, dst_ref=x_all.at[me],
                send_sem=xsend_sems.at[t], recv_sem=xrecv_sems.at[me],
                device_id=((me + t) % N_DEV,),
                device_id_type=pl.DeviceIdType.MESH,
            ).wait_send()

    out, _, _ = pl.pallas_call(
        body,
        out_shape=[
            jax.ShapeDtypeStruct((1, SQ, D), BF16),
            jax.ShapeDtypeStruct((N_DEV, SQ, D), BF16),
            jax.ShapeDtypeStruct((N_DEV, SQ, D), BF16),
        ],
        in_specs=[pl.BlockSpec(memory_space=pltpu.VMEM)] * 7,
        out_specs=[
            pl.BlockSpec(memory_space=pltpu.VMEM),
            pl.BlockSpec(memory_space=pl.ANY),
            pl.BlockSpec(memory_space=pl.ANY),
        ],
        scratch_shapes=[
            pltpu.VMEM((SQ, D), BF16),
            pltpu.VMEM((2, SQ, D), BF16),
            pltpu.VMEM((SQ, D), BF16),
            pltpu.VMEM((SQ, D), BF16),
            pltpu.VMEM((SQ, D), F32),
            pltpu.VMEM((SQ, D), F32),
            pltpu.VMEM((SQ, D), BF16),
            pltpu.VMEM((SQ, D), F32),
            pltpu.SemaphoreType.DMA((N_DEV,)),
            pltpu.SemaphoreType.DMA((N_DEV,)),
            pltpu.SemaphoreType.DMA((2,)),
            pltpu.SemaphoreType.DMA((N_DEV,)),
            pltpu.SemaphoreType.DMA,
        ],
        compiler_params=pltpu.CompilerParams(
            collective_id=0, vmem_limit_bytes=40 * 1024 * 1024),
    )(x2, wq, wk, wv, wo, cos, sin)
    return out
